# baseline (speedup 1.0000x reference)
import functools

import jax
import jax.numpy as jnp
from jax import lax
from jax.experimental import pallas as pl
from jax.experimental.pallas import tpu as pltpu
from jax.experimental.pallas import tpu_sc as plsc

E = 8
NBUF = 8
SC_N = 512
NW = 32
L = 16


def _tc_body(gl_ref, w_hbm, hc_ref, x_ref, win_ref,
             o_ref, wbuf, acc_ref, idx_ref, zs_ref, sems, *, bond, split):
    def scan(j, p):
        g = gl_ref[j]
        z = jnp.minimum(jnp.maximum(g, 0.0), 1.0)
        idx_ref[p] = j
        zs_ref[p] = z
        return p + (z > 0.0).astype(jnp.int32)

    na = jax.lax.fori_loop(0, split, scan, 0, unroll=False)

    lastj = jnp.where(na > 0, idx_ref[jnp.maximum(na - 1, 0)], 0)
    for e in range(E):
        idx_ref[na + e] = lastj
        zs_ref[na + e] = 0.0

    nsteps = (na + E - 1) // E

    def issue(step, slot):
        for e in range(E):
            j = idx_ref[step * E + e]
            pltpu.make_async_copy(
                w_hbm.at[pl.ds(j, 1)], wbuf.at[slot, pl.ds(e, 1)],
                sems.at[slot]).start()

    def wait(step, slot):
        for e in range(E):
            j = idx_ref[step * E + e]
            pltpu.make_async_copy(
                w_hbm.at[pl.ds(j, 1)], wbuf.at[slot, pl.ds(e, 1)],
                sems.at[slot]).wait()

    acc_ref[...] = jnp.zeros_like(acc_ref)

    for k in range(NBUF - 1):
        @pl.when(k < nsteps)
        def _prologue():
            issue(k, k)

    def loop(step, carry):
        slot = jax.lax.rem(step, NBUF)

        @pl.when(step + NBUF - 1 < nsteps)
        def _next():
            issue(step + NBUF - 1, jax.lax.rem(step + NBUF - 1, NBUF))

        wait(step, slot)
        partial = jnp.zeros((1, bond), jnp.float32)
        for e in range(E):
            k = step * E + e
            j = idx_ref[k]
            zv = zs_ref[k]
            y = hc_ref[pl.ds(j, 1), :] * zv
            partial += jax.lax.dot_general(
                y, wbuf[slot, e], (((1,), (1,)), ((), ())),
                preferred_element_type=jnp.float32)
        acc_ref[...] += partial
        return carry

    jax.lax.fori_loop(0, nsteps, loop, 0, unroll=False)

    h = jax.lax.dot_general(
        x_ref[...], win_ref[...], (((1,), (1,)), ((), ())),
        preferred_element_type=jnp.float32)
    o_ref[...] = h + acc_ref[...]


def _sc_body(gl_hbm, hc_hbm, w_hbm, out_hbm, glbuf, ybuf, wbuf, tacc, sem,
             *, bond, split):
    chunk = SC_N // NW
    QB = 4
    qrows = bond // QB
    kpr = 1024 // bond
    blocks = chunk * QB
    wid = lax.axis_index("s") * 2 + lax.axis_index("c")
    start = split + wid * chunk

    pltpu.sync_copy(gl_hbm, glbuf.at[pl.ds(0, glbuf.shape[0] - L)])
    pltpu.sync_copy(hc_hbm.at[pl.ds(start, chunk)], ybuf)

    def zinit(k, c):
        tacc[k] = jnp.zeros((L,), jnp.float32)
        return c

    lax.fori_loop(0, bond, zinit, 0, unroll=False)

    def issue(b, slot):
        e = b // QB
        q = lax.rem(b, QB)
        j = start + e
        idxv = jnp.full((L,), j * 64 + q * L, jnp.int32) + lax.iota(jnp.int32, L)
        pltpu.make_async_copy(
            w_hbm.at[idxv], wbuf.at[slot], sem.at[slot]).start()

    def wait_for(b, slot):
        e = b // QB
        q = lax.rem(b, QB)
        j = start + e
        idxv = jnp.full((L,), j * 64 + q * L, jnp.int32) + lax.iota(jnp.int32, L)
        pltpu.make_async_copy(
            w_hbm.at[idxv], wbuf.at[slot], sem.at[slot]).wait()

    for k in range(QB - 1):
        issue(k, k)

    def loop(b, carry):
        slot = lax.rem(b, QB)

        @pl.when(b + QB - 1 < blocks)
        def _next():
            issue(b + QB - 1, lax.rem(b + QB - 1, QB))

        e = b // QB
        q = lax.rem(b, QB)
        j = start + e
        wait_for(b, slot)

        g = glbuf[pl.ds(j, L)][0]
        zsp = jnp.full((L,), jnp.minimum(jnp.maximum(g, 0.0), 1.0), jnp.float32)
        yv = [ybuf[e, pl.ds(l * L, L)] * zsp for l in range(bond // L)]

        def krow(r, c):
            for kk in range(kpr):
                acc = jnp.zeros((L,), jnp.float32)
                for l in range(bond // L):
                    acc = acc + wbuf[slot, r, pl.ds(kk * bond + l * L, L)] * yv[l]
                plsc.addupdate(tacc.at[q * qrows + r * kpr + kk], acc)
            return c

        lax.fori_loop(0, L, krow, 0, unroll=2)
        return carry

    lax.fori_loop(0, blocks, loop, 0, unroll=False)

    pltpu.sync_copy(tacc, out_hbm.at[wid])


def _fin_body(v_ref, sc_ref, wout_ref, o_ref, *, bond):
    t = jnp.sum(sc_ref[...], axis=0)
    ones = jnp.ones((1, L), jnp.float32)
    v_sc = jax.lax.dot_general(
        ones, t, (((1,), (1,)), ((), ())),
        preferred_element_type=jnp.float32)
    outv = v_ref[...] + v_sc
    o_ref[...] = jax.lax.dot_general(
        outv, wout_ref[...], (((1,), (1,)), ((), ())),
        preferred_element_type=jnp.float32)


def kernel(x, pos, hidden_cache, W_in, W_out, W_edges, gate_logits):
    del pos
    POS, BOND = hidden_cache.shape
    N = x.shape[0]
    SPLIT = POS - SC_N

    v_tc = pl.pallas_call(
        functools.partial(_tc_body, bond=BOND, split=SPLIT),
        in_specs=[
            pl.BlockSpec(memory_space=pltpu.SMEM),
            pl.BlockSpec(memory_space=pl.ANY),
            pl.BlockSpec(memory_space=pltpu.VMEM),
            pl.BlockSpec(memory_space=pltpu.VMEM),
            pl.BlockSpec(memory_space=pltpu.VMEM),
        ],
        out_specs=pl.BlockSpec(memory_space=pltpu.VMEM),
        out_shape=jax.ShapeDtypeStruct((1, BOND), jnp.float32),
        scratch_shapes=[
            pltpu.VMEM((NBUF, E, BOND, BOND), jnp.float32),
            pltpu.VMEM((1, BOND), jnp.float32),
            pltpu.SMEM((POS + E,), jnp.int32),
            pltpu.SMEM((POS + E,), jnp.float32),
            pltpu.SemaphoreType.DMA((NBUF,)),
        ],
    )(gate_logits, W_edges, hidden_cache, x.reshape(1, N), W_in)

    sc_kernel = pl.kernel(
        functools.partial(_sc_body, bond=BOND, split=SPLIT),
        out_type=jax.ShapeDtypeStruct((NW, BOND, L), jnp.float32),
        mesh=plsc.VectorSubcoreMesh(core_axis_name="c", subcore_axis_name="s"),
        scratch_types=[
            pltpu.VMEM((POS + L,), jnp.float32),
            pltpu.VMEM((SC_N // NW, BOND), jnp.float32),
            pltpu.VMEM((4, L, 1024), jnp.float32),
            pltpu.VMEM((BOND, L), jnp.float32),
            pltpu.SemaphoreType.DMA((4,)),
        ],
    )
    sc_part = sc_kernel(gate_logits, hidden_cache,
                        W_edges.reshape(POS * 64, 1024))

    out = pl.pallas_call(
        functools.partial(_fin_body, bond=BOND),
        in_specs=[
            pl.BlockSpec(memory_space=pltpu.VMEM),
            pl.BlockSpec(memory_space=pltpu.VMEM),
            pl.BlockSpec(memory_space=pltpu.VMEM),
        ],
        out_specs=pl.BlockSpec(memory_space=pltpu.VMEM),
        out_shape=jax.ShapeDtypeStruct((1, N), jnp.float32),
    )(v_tc, sc_part, W_out)
    return out.reshape(N)


# speedup vs baseline: 5.2666x; 5.2666x over previous
import functools

import jax
import jax.numpy as jnp
from jax import lax
from jax.experimental import pallas as pl
from jax.experimental.pallas import tpu as pltpu
from jax.experimental.pallas import tpu_sc as plsc

E = 8
NBUF = 8
SC_N = 352
NW = 32
L = 16


def _tc_body(gl_ref, w_hbm, hc_ref, x_ref, win_ref,
             o_ref, wbuf, acc_ref, idx_ref, zs_ref, sems, *, bond, lo, hi):
    def scan(j, p):
        g = gl_ref[j]
        z = jnp.minimum(jnp.maximum(g, 0.0), 1.0)
        idx_ref[p] = j
        zs_ref[p] = z
        return p + (z > 0.0).astype(jnp.int32)

    na = jax.lax.fori_loop(lo, hi, scan, 0, unroll=False)

    lastj = jnp.where(na > 0, idx_ref[jnp.maximum(na - 1, 0)], 0)
    for e in range(E):
        idx_ref[na + e] = lastj
        zs_ref[na + e] = 0.0

    nsteps = (na + E - 1) // E

    def issue(step, slot):
        for e in range(E):
            j = idx_ref[step * E + e]
            pltpu.make_async_copy(
                w_hbm.at[pl.ds(j, 1)], wbuf.at[slot, pl.ds(e, 1)],
                sems.at[slot]).start()

    def wait(step, slot):
        for e in range(E):
            j = idx_ref[step * E + e]
            pltpu.make_async_copy(
                w_hbm.at[pl.ds(j, 1)], wbuf.at[slot, pl.ds(e, 1)],
                sems.at[slot]).wait()

    acc_ref[...] = jnp.zeros_like(acc_ref)

    for k in range(NBUF - 1):
        @pl.when(k < nsteps)
        def _prologue():
            issue(k, k)

    def loop(step, carry):
        slot = jax.lax.rem(step, NBUF)

        @pl.when(step + NBUF - 1 < nsteps)
        def _next():
            issue(step + NBUF - 1, jax.lax.rem(step + NBUF - 1, NBUF))

        wait(step, slot)
        partial = jnp.zeros((1, bond), jnp.float32)
        for e in range(E):
            k = step * E + e
            j = idx_ref[k]
            zv = zs_ref[k]
            y = hc_ref[pl.ds(j, 1), :] * zv
            partial += jax.lax.dot_general(
                y, wbuf[slot, e], (((1,), (1,)), ((), ())),
                preferred_element_type=jnp.float32)
        acc_ref[...] += partial
        return carry

    jax.lax.fori_loop(0, nsteps, loop, 0, unroll=False)

    h = jax.lax.dot_general(
        x_ref[...], win_ref[...], (((1,), (1,)), ((), ())),
        preferred_element_type=jnp.float32)
    o_ref[...] = h + acc_ref[...]


def _sc_body(gl_hbm, hc_hbm, w_hbm, out_hbm, glbuf, ybuf, wbuf, tacc, sem,
             *, bond):
    chunk = SC_N // NW
    QB = 4
    qrows = bond // QB
    blocks = chunk * QB
    wid = lax.axis_index("s") * 2 + lax.axis_index("c")
    start = wid * chunk

    pltpu.sync_copy(gl_hbm, glbuf.at[pl.ds(0, glbuf.shape[0] - L)])
    astart = (start // 8) * 8
    aoff = start - astart
    pltpu.sync_copy(hc_hbm.at[pl.ds(astart, ybuf.shape[0])], ybuf)

    def zinit(k, c):
        tacc[k] = jnp.zeros((L,), jnp.float32)
        return c

    lax.fori_loop(0, bond, zinit, 0, unroll=False)

    def issue(b, slot):
        e = b // QB
        q = lax.rem(b, QB)
        j = start + e
        pltpu.make_async_copy(
            w_hbm.at[j, pl.ds(q * qrows, qrows)], wbuf.at[slot],
            sem.at[slot]).start()

    for k in range(QB - 1):
        issue(k, k)

    def loop(b, carry):
        slot = lax.rem(b, QB)

        @pl.when(b + QB - 1 < blocks)
        def _next():
            issue(b + QB - 1, lax.rem(b + QB - 1, QB))

        e = b // QB
        q = lax.rem(b, QB)
        j = start + e
        pltpu.make_async_copy(
            w_hbm.at[j, pl.ds(q * qrows, qrows)], wbuf.at[slot],
            sem.at[slot]).wait()

        g = glbuf[pl.ds(j, L)][0]
        zsp = jnp.full((L,), jnp.minimum(jnp.maximum(g, 0.0), 1.0), jnp.float32)
        yv = [ybuf[e + aoff, pl.ds(l * L, L)] * zsp for l in range(bond // L)]

        def krow(k, c):
            acc = jnp.zeros((L,), jnp.float32)
            for l in range(bond // L):
                acc = acc + wbuf[slot, k, pl.ds(l * L, L)] * yv[l]
            plsc.addupdate(tacc.at[q * qrows + k], acc)
            return c

        lax.fori_loop(0, qrows, krow, 0, unroll=4)
        return carry

    lax.fori_loop(0, blocks, loop, 0, unroll=False)

    pltpu.sync_copy(tacc, out_hbm.at[wid])


def _fin_body(v_ref, sc_ref, wout_ref, o_ref, *, bond):
    t = jnp.sum(sc_ref[...], axis=0)
    ones = jnp.ones((1, L), jnp.float32)
    v_sc = jax.lax.dot_general(
        ones, t, (((1,), (1,)), ((), ())),
        preferred_element_type=jnp.float32)
    outv = v_ref[...] + v_sc
    o_ref[...] = jax.lax.dot_general(
        outv, wout_ref[...], (((1,), (1,)), ((), ())),
        preferred_element_type=jnp.float32)


def kernel(x, pos, hidden_cache, W_in, W_out, W_edges, gate_logits):
    del pos
    POS, BOND = hidden_cache.shape
    N = x.shape[0]

    v_tc = pl.pallas_call(
        functools.partial(_tc_body, bond=BOND, lo=SC_N, hi=POS),
        in_specs=[
            pl.BlockSpec(memory_space=pltpu.SMEM),
            pl.BlockSpec(memory_space=pl.ANY),
            pl.BlockSpec(memory_space=pltpu.VMEM),
            pl.BlockSpec(memory_space=pltpu.VMEM),
            pl.BlockSpec(memory_space=pltpu.VMEM),
        ],
        out_specs=pl.BlockSpec(memory_space=pltpu.VMEM),
        out_shape=jax.ShapeDtypeStruct((1, BOND), jnp.float32),
        scratch_shapes=[
            pltpu.VMEM((NBUF, E, BOND, BOND), jnp.float32),
            pltpu.VMEM((1, BOND), jnp.float32),
            pltpu.SMEM((POS + E,), jnp.int32),
            pltpu.SMEM((POS + E,), jnp.float32),
            pltpu.SemaphoreType.DMA((NBUF,)),
        ],
    )(gate_logits, W_edges, hidden_cache, x.reshape(1, N), W_in)

    sc_kernel = pl.kernel(
        functools.partial(_sc_body, bond=BOND),
        out_type=jax.ShapeDtypeStruct((NW, BOND, L), jnp.float32),
        mesh=plsc.VectorSubcoreMesh(core_axis_name="c", subcore_axis_name="s"),
        scratch_types=[
            pltpu.VMEM((POS + L,), jnp.float32),
            pltpu.VMEM(((SC_N // NW + 7) // 8 * 8 + 8, BOND), jnp.float32),
            pltpu.VMEM((4, BOND // 4, BOND), jnp.float32),
            pltpu.VMEM((BOND, L), jnp.float32),
            pltpu.SemaphoreType.DMA((4,)),
        ],
    )
    sc_part = sc_kernel(gate_logits, hidden_cache, W_edges)

    out = pl.pallas_call(
        functools.partial(_fin_body, bond=BOND),
        in_specs=[
            pl.BlockSpec(memory_space=pltpu.VMEM),
            pl.BlockSpec(memory_space=pltpu.VMEM),
            pl.BlockSpec(memory_space=pltpu.VMEM),
        ],
        out_specs=pl.BlockSpec(memory_space=pltpu.VMEM),
        out_shape=jax.ShapeDtypeStruct((1, N), jnp.float32),
    )(v_tc, sc_part, W_out)
    return out.reshape(N)


# speedup vs baseline: 6.5874x; 1.2508x over previous
import functools

import jax
import jax.numpy as jnp
from jax.experimental import pallas as pl
from jax.experimental.pallas import tpu as pltpu

E = 8
NBUF = 8


def _body(gl_ref, w_hbm, hc_ref, x_ref, win_ref, wout_ref,
          o_ref, wbuf, acc_ref, idx_ref, zs_ref, sems, *, bond, n_embd, pos):
    def scan(j, p):
        g = gl_ref[j]
        z = jnp.minimum(jnp.maximum(g, 0.0), 1.0)
        idx_ref[p] = j
        zs_ref[p] = z
        return p + (z > 0.0).astype(jnp.int32)

    na = jax.lax.fori_loop(0, pos, scan, 0, unroll=False)

    lastj = jnp.where(na > 0, idx_ref[jnp.maximum(na - 1, 0)], 0)
    for e in range(E):
        idx_ref[na + e] = lastj
        zs_ref[na + e] = 0.0

    nsteps = (na + E - 1) // E

    def issue(step, slot):
        for e in range(E):
            j = idx_ref[step * E + e]
            pltpu.make_async_copy(
                w_hbm.at[pl.ds(j, 1)], wbuf.at[slot, pl.ds(e, 1)],
                sems.at[slot]).start()

    def wait(step, slot):
        for e in range(E):
            j = idx_ref[step * E + e]
            pltpu.make_async_copy(
                w_hbm.at[pl.ds(j, 1)], wbuf.at[slot, pl.ds(e, 1)],
                sems.at[slot]).wait()

    acc_ref[...] = jnp.zeros_like(acc_ref)

    for k in range(NBUF - 1):
        @pl.when(k < nsteps)
        def _prologue():
            issue(k, k)

    def loop(step, carry):
        slot = jax.lax.rem(step, NBUF)

        @pl.when(step + NBUF - 1 < nsteps)
        def _next():
            issue(step + NBUF - 1, jax.lax.rem(step + NBUF - 1, NBUF))

        wait(step, slot)
        partial = jnp.zeros((1, bond), jnp.float32)
        for e in range(E):
            k = step * E + e
            j = idx_ref[k]
            zv = zs_ref[k]
            y = hc_ref[pl.ds(j, 1), :] * zv
            partial += jax.lax.dot_general(
                y, wbuf[slot, e], (((1,), (1,)), ((), ())),
                preferred_element_type=jnp.float32)
        acc_ref[...] += partial
        return carry

    jax.lax.fori_loop(0, nsteps, loop, 0, unroll=False)

    h = jax.lax.dot_general(
        x_ref[...], win_ref[...], (((1,), (1,)), ((), ())),
        preferred_element_type=jnp.float32)
    outv = h + acc_ref[...]
    o_ref[...] = jax.lax.dot_general(
        outv, wout_ref[...], (((1,), (1,)), ((), ())),
        preferred_element_type=jnp.float32)


def kernel(x, pos, hidden_cache, W_in, W_out, W_edges, gate_logits):
    del pos
    POS, BOND = hidden_cache.shape
    N = x.shape[0]

    out = pl.pallas_call(
        functools.partial(_body, bond=BOND, n_embd=N, pos=POS),
        in_specs=[
            pl.BlockSpec(memory_space=pltpu.SMEM),
            pl.BlockSpec(memory_space=pl.ANY),
            pl.BlockSpec(memory_space=pltpu.VMEM),
            pl.BlockSpec(memory_space=pltpu.VMEM),
            pl.BlockSpec(memory_space=pltpu.VMEM),
            pl.BlockSpec(memory_space=pltpu.VMEM),
        ],
        out_specs=pl.BlockSpec(memory_space=pltpu.VMEM),
        out_shape=jax.ShapeDtypeStruct((1, N), jnp.float32),
        scratch_shapes=[
            pltpu.VMEM((NBUF, E, BOND, BOND), jnp.float32),
            pltpu.VMEM((1, BOND), jnp.float32),
            pltpu.SMEM((POS + E,), jnp.int32),
            pltpu.SMEM((POS + E,), jnp.float32),
            pltpu.SemaphoreType.DMA((NBUF,)),
        ],
    )(gate_logits, W_edges, hidden_cache, x.reshape(1, N), W_in, W_out)
    return out.reshape(N)
